# Initial kernel scaffold; baseline (speedup 1.0000x reference)
import jax
import jax.numpy as jnp
from jax.experimental import pallas as pl


def kernel(z_e, emb):
    raise NotImplementedError("write your pallas kernel here")


# speedup vs baseline: 1.5463x; 1.5463x over previous
import jax
import jax.numpy as jnp
from jax.experimental import pallas as pl
from jax.experimental.pallas import tpu as pltpu

_NUM_CODES = 512
_CDIM = 64
_PIX_TILE = 1024
_BETA = 0.25


def _vq_body(x_ref, embT_ref, emb_ref, zq_ref, codes_ref, loss_ref):
    x = x_ref[...]
    embT = embT_ref[...]
    emb = emb_ref[...]

    xsq = jnp.sum(x * x, axis=1, keepdims=True)
    esq = jnp.sum(emb * emb, axis=1)
    m = jnp.dot(x, embT, preferred_element_type=jnp.float32)
    dist = (xsq + esq[None, :]) - 2.0 * m

    mn = jnp.min(dist, axis=1, keepdims=True)
    lane = jax.lax.broadcasted_iota(jnp.int32, dist.shape, 1)
    codes = jnp.min(jnp.where(dist == mn, lane, _NUM_CODES), axis=1)
    codes_ref[0, 0, :] = codes

    onehot = (lane == codes[:, None]).astype(jnp.float32)
    zq = jax.lax.dot_general(
        onehot, emb, (((1,), (0,)), ((), ())),
        precision=jax.lax.Precision.HIGHEST,
        preferred_element_type=jnp.float32)

    zq_ref[0] = x + (zq - x)
    loss_ref[0, 0, 0] = jnp.sum((zq - x) ** 2)


def kernel(z_e, emb):
    B, C, H, W = z_e.shape
    P = B * H * W
    z_flat = jnp.transpose(z_e, (0, 2, 3, 1)).reshape(P, C)
    grid = P // _PIX_TILE

    zq_st, codes3, lossp = pl.pallas_call(
        _vq_body,
        grid=(grid,),
        in_specs=[
            pl.BlockSpec((_PIX_TILE, C), lambda m: (m, 0)),
            pl.BlockSpec((C, _NUM_CODES), lambda m: (0, 0)),
            pl.BlockSpec((_NUM_CODES, C), lambda m: (0, 0)),
        ],
        out_specs=[
            pl.BlockSpec((1, _PIX_TILE, C), lambda m: (m, 0, 0)),
            pl.BlockSpec((1, 1, _PIX_TILE), lambda m: (m, 0, 0)),
            pl.BlockSpec((1, 1, 1), lambda m: (m, 0, 0),
                         memory_space=pltpu.SMEM),
        ],
        out_shape=[
            jax.ShapeDtypeStruct((grid, _PIX_TILE, C), jnp.float32),
            jax.ShapeDtypeStruct((grid, 1, _PIX_TILE), jnp.int32),
            jax.ShapeDtypeStruct((grid, 1, 1), jnp.float32),
        ],
    )(z_flat, emb.T, emb)

    zq_st = zq_st.reshape(B, H, W, C).transpose(0, 3, 1, 2)
    codes = codes3.reshape(B, H, W)
    vq_loss = (1.0 + _BETA) * jnp.sum(lossp) / (B * C * H * W)
    return zq_st, vq_loss, codes
